# baseline (speedup 1.0000x reference)
import functools

import jax
import jax.numpy as jnp
from jax import lax
from jax.experimental import pallas as pl
from jax.experimental.pallas import tpu as pltpu
from jax.experimental.pallas import tpu_sc as plsc

B = 4
D = 8
C = 8
N = 512 * 512
NC = 2
NS = 16
L = 16
BPC = B // NC
PPT = N // NS
CH = 4096
W = 512
RPC = CH // W
NCHUNK = PPT // CH
NV = CH // L
NITEM = 2 * BPC * NCHUNK

DELTA_VAR = 1.0
DELTA_DIST = 2.0

_MAGIC = 0x5F3759DF


def _safe_norm(sq):
    seed = jnp.int32(_MAGIC) - (plsc.bitcast(sq, jnp.int32) >> 1)
    y = plsc.bitcast(seed, jnp.float32)
    for _ in range(3):
        y = y * (1.5 - 0.5 * sq * y * y)
    return sq * y


def _recip_s(s):
    return (1.0 / jnp.broadcast_to(s, (L,)))[0]


def _pack(scalars, iota):
    v = jnp.zeros((L,), jnp.float32)
    for j, s in enumerate(scalars):
        v = jnp.where(iota == j, s, v)
    return v


def _sc_loss_fn():
    mesh = plsc.VectorSubcoreMesh(core_axis_name="c", subcore_axis_name="s",
                                  num_cores=NC)

    @functools.partial(
        pl.kernel,
        out_type=jax.ShapeDtypeStruct((NC * L,), jnp.float32),
        mesh=mesh,
        compiler_params=pltpu.CompilerParams(needs_layout_passes=False),
        scratch_types=[
            pltpu.VMEM((2, D, RPC, W), jnp.float32),
            pltpu.VMEM((2, RPC, W), jnp.int32),
            pltpu.VMEM((D * C * L,), jnp.float32),
            pltpu.VMEM((C * L,), jnp.float32),
            pltpu.VMEM((96,), jnp.float32),
            pltpu.VMEM((NS * 96,), jnp.float32),
            pltpu.VMEM((BPC * D * L,), jnp.float32),
            pltpu.VMEM((BPC * L,), jnp.float32),
            pltpu.VMEM((BPC * L,), jnp.float32),
            pltpu.VMEM((L,), jnp.float32),
            pltpu.VMEM((NS * L,), jnp.float32),
            pltpu.VMEM((L,), jnp.float32),
            pltpu.VMEM_SHARED((BPC * NS * 96,), jnp.float32),
            pltpu.VMEM_SHARED((BPC * NS * L,), jnp.float32),
            pltpu.SemaphoreType.DMA,
            pltpu.SemaphoreType.DMA,
        ],
    )
    def body(data_hbm, labels_hbm, out_hbm,
             dbuf, lbuf, sumtbl, cnttbl, stage, aggbuf,
             ctr_tbl, inv_tbl, cntv_tbl, vstage, varbuf, ostage,
             sh_stats, sh_var, sem0, sem1):
        cid = lax.axis_index("c")
        sid = lax.axis_index("s")
        iota = lax.iota(jnp.int32, L)
        zeros = jnp.zeros((L,), jnp.float32)
        ones = jnp.ones((L,), jnp.float32)
        sems = [sem0, sem1]

        def _start(i):
            slot = i % 2
            b = (i % (BPC * NCHUNK)) // NCHUNK
            ch = i % NCHUNK
            gb = cid * BPC + b
            row0 = pl.multiple_of((sid * PPT + ch * CH) // W, 8)
            h1 = pltpu.async_copy(data_hbm.at[gb, :, pl.ds(row0, RPC), :],
                                  dbuf.at[slot], sems[slot])
            h2 = pltpu.async_copy(labels_hbm.at[gb, pl.ds(row0, RPC), :],
                                  lbuf.at[slot], sems[slot])
            return (h1, h2)

        pending = [None] * NITEM
        pending[0] = _start(0)
        pending[1] = _start(1)

        def _finish(i):
            h1, h2 = pending[i]
            h1.wait()
            h2.wait()
            return i % 2

        for b in range(BPC):
            def _z_sum(r, _):
                sumtbl[pl.ds(r * L, L)] = zeros
                return 0
            lax.fori_loop(0, D * C, _z_sum, 0)

            def _z_cnt(r, _):
                cnttbl[pl.ds(r * L, L)] = zeros
                return 0
            lax.fori_loop(0, C, _z_cnt, 0)

            for ch in range(NCHUNK):
                i = b * NCHUNK + ch
                slot = _finish(i)

                @plsc.parallel_loop(0, NV, 1, unroll=4)
                def _vec1(v, slot=slot):
                    r = v // (W // L)
                    col = (v % (W // L)) * L
                    lab = lbuf[slot, r, pl.ds(col, L)]
                    idx = lab * L + iota
                    plsc.addupdate_scatter(cnttbl, [idx], ones)
                    for dim in range(D):
                        x = dbuf[slot, dim, r, pl.ds(col, L)]
                        plsc.addupdate_scatter(sumtbl,
                                               [idx + dim * (C * L)], x)
                if i + 2 < NITEM:
                    pending[i + 2] = _start(i + 2)

            scalars = [jnp.sum(sumtbl[pl.ds(r * L, L)]) for r in range(D * C)]
            scalars += [jnp.sum(cnttbl[pl.ds(c * L, L)]) for c in range(C)]
            for j in range(6):
                stage[pl.ds(j * L, L)] = _pack(scalars[j * L:(j + 1) * L], iota)
            pltpu.sync_copy(stage, sh_stats.at[pl.ds((b * NS + sid) * 96, 96)])

        plsc.subcore_barrier()

        k_list = []
        active_list = []
        pres_list = []
        for b in range(BPC):
            pltpu.sync_copy(sh_stats.at[pl.ds(b * NS * 96, NS * 96)], aggbuf)
            aggv = []
            for j in range(6):
                acc = zeros
                for t in range(NS):
                    acc = acc + aggbuf[pl.ds(t * 96 + j * L, L)]
                aggv.append(acc)

            csv = aggv[4]
            pres_vec = jnp.where(csv > 0.0, 1.0, 0.0)
            inv_vec = 1.0 / jnp.where(csv > 0.0, csv, 1.0)
            k = jnp.sum(pres_vec)
            active = jnp.where(k > 1.0, 1.0, 0.0)
            k_list.append(k)
            active_list.append(active)
            pres_list.append([pres_vec[c] for c in range(C)])

            inv_tbl[pl.ds(b * L, L)] = inv_vec
            cntv_tbl[pl.ds(b * L, L)] = csv
            inv = [inv_vec[c] for c in range(C)]
            for dim in range(D):
                ctr = [aggv[(dim * C + c) // L][(dim * C + c) % L] * inv[c]
                       for c in range(C)]
                ctr_tbl[pl.ds(b * (D * L) + dim * L, L)] = _pack(ctr, iota)

        for b in range(BPC):
            cbase = b * (D * L)

            def _z_var(r, _):
                cnttbl[pl.ds(r * L, L)] = zeros
                return 0
            lax.fori_loop(0, C, _z_var, 0)

            for ch in range(NCHUNK):
                i = BPC * NCHUNK + b * NCHUNK + ch
                slot = _finish(i)

                csel = []
                for dim in range(D - 2, D):
                    row = ctr_tbl[pl.ds(cbase + dim * L, L)]
                    csel.append([jnp.broadcast_to(row[c], (L,))
                                 for c in range(C)])

                @plsc.parallel_loop(0, NV, 1, unroll=4)
                def _vec2(v, slot=slot, cbase=cbase, csel=csel):
                    r = v // (W // L)
                    col = (v % (W // L)) * L
                    lab = lbuf[slot, r, pl.ds(col, L)]
                    idx = lab * L + iota
                    dsq = jnp.zeros((L,), jnp.float32)
                    for dim in range(D - 2):
                        x = dbuf[slot, dim, r, pl.ds(col, L)]
                        cv = plsc.load_gather(ctr_tbl,
                                              [lab + (cbase + dim * L)])
                        dd = x - cv
                        dsq = dsq + dd * dd
                    for j, dim in enumerate(range(D - 2, D)):
                        x = dbuf[slot, dim, r, pl.ds(col, L)]
                        cv = csel[j][0]
                        for c in range(1, C):
                            cv = jnp.where(lab == c, csel[j][c], cv)
                        dd = x - cv
                        dsq = dsq + dd * dd
                    norm = _safe_norm(dsq)
                    t = jnp.maximum(norm - DELTA_VAR, 0.0)
                    plsc.addupdate_scatter(cnttbl, [idx], t * t)
                if i + 2 < NITEM:
                    pending[i + 2] = _start(i + 2)

            vstage[...] = _pack([jnp.sum(cnttbl[pl.ds(c * L, L)])
                                 for c in range(C)], iota)
            pltpu.sync_copy(vstage, sh_var.at[pl.ds((b * NS + sid) * L, L)])

        plsc.subcore_barrier()

        @pl.when(sid == 0)
        def _final():
            total = jnp.float32(0.0)
            for b in range(BPC):
                pltpu.sync_copy(sh_var.at[pl.ds(b * NS * L, NS * L)], varbuf)
                vsum = zeros
                for t in range(NS):
                    vsum = vsum + varbuf[pl.ds(t * L, L)]
                var_sum = jnp.sum(vsum * inv_tbl[pl.ds(b * L, L)])

                cbase = b * (D * L)
                cntv = cntv_tbl[pl.ds(b * L, L)]
                pres_vec = jnp.where(cntv > 0.0, 1.0, 0.0)
                k = k_list[b]
                pres = pres_list[b]

                dist_acc = jnp.float32(0.0)
                for ci in range(C):
                    dsq = jnp.zeros((L,), jnp.float32)
                    for dim in range(D):
                        cv = ctr_tbl[pl.ds(cbase + dim * L, L)]
                        dd = cv - jnp.broadcast_to(cv[ci], (L,))
                        dsq = dsq + dd * dd
                    dnorm = _safe_norm(dsq)
                    t = jnp.maximum(DELTA_DIST - dnorm, 0.0)
                    dist_acc = dist_acc + jnp.sum(t * t * pres_vec) * pres[ci]
                safe_k = jnp.where(k > 1.0, k, 2.0)
                dist_sum = dist_acc * _recip_s(2.0 * safe_k * (safe_k - 1.0))

                rsq = jnp.zeros((L,), jnp.float32)
                for dim in range(D):
                    cv = ctr_tbl[pl.ds(cbase + dim * L, L)]
                    rsq = rsq + cv * cv
                reg_sum = jnp.sum(_safe_norm(rsq) * pres_vec)
                reg_sum = reg_sum * _recip_s(jnp.where(k > 0.0, k, 1.0))

                total = total + active_list[b] * (var_sum + dist_sum + reg_sum)

            ostage[...] = jnp.broadcast_to(total, (L,))
            pltpu.sync_copy(ostage, out_hbm.at[pl.ds(cid * L, L)])

    return body


_sc_loss = jax.jit(_sc_loss_fn())


def kernel(data, labels):
    out = _sc_loss(data, labels.astype(jnp.int32))
    return (out[0] + out[L]) * (1.0 / B)


# speedup vs baseline: 1.0964x; 1.0964x over previous
import functools

import jax
import jax.numpy as jnp
from jax import lax
from jax.experimental import pallas as pl
from jax.experimental.pallas import tpu as pltpu
from jax.experimental.pallas import tpu_sc as plsc

B = 4
D = 8
C = 8
N = 512 * 512
NC = 2
NS = 16
L = 16
BPC = B // NC
PPT = N // NS
CH = 4096
W = 512
RPC = CH // W
NCHUNK = PPT // CH
NV = CH // L
NITEM = 2 * BPC * NCHUNK

DELTA_VAR = 1.0
DELTA_DIST = 2.0

_MAGIC = 0x5F3759DF


def _safe_norm(sq):
    seed = jnp.int32(_MAGIC) - (plsc.bitcast(sq, jnp.int32) >> 1)
    y = plsc.bitcast(seed, jnp.float32)
    for _ in range(3):
        y = y * (1.5 - 0.5 * sq * y * y)
    return sq * y


def _recip_s(s):
    return (1.0 / jnp.broadcast_to(s, (L,)))[0]


def _pack(scalars, iota):
    v = jnp.zeros((L,), jnp.float32)
    for j, s in enumerate(scalars):
        v = jnp.where(iota == j, s, v)
    return v


def _sc_loss_fn():
    mesh = plsc.VectorSubcoreMesh(core_axis_name="c", subcore_axis_name="s",
                                  num_cores=NC)

    @functools.partial(
        pl.kernel,
        out_type=jax.ShapeDtypeStruct((NC * L,), jnp.float32),
        mesh=mesh,
        compiler_params=pltpu.CompilerParams(needs_layout_passes=False),
        scratch_types=[
            pltpu.VMEM((2, D, RPC, W), jnp.float32),
            pltpu.VMEM((2, RPC, W), jnp.int32),
            pltpu.VMEM((D * C * L,), jnp.float32),
            pltpu.VMEM((C * L,), jnp.float32),
            pltpu.VMEM((96,), jnp.float32),
            pltpu.VMEM((NS * 96,), jnp.float32),
            pltpu.VMEM((BPC * D * L,), jnp.float32),
            pltpu.VMEM((BPC * L,), jnp.float32),
            pltpu.VMEM((BPC * L,), jnp.float32),
            pltpu.VMEM((L,), jnp.float32),
            pltpu.VMEM((NS * L,), jnp.float32),
            pltpu.VMEM((L,), jnp.float32),
            pltpu.VMEM_SHARED((BPC * NS * 96,), jnp.float32),
            pltpu.VMEM_SHARED((BPC * NS * L,), jnp.float32),
            pltpu.SemaphoreType.DMA,
            pltpu.SemaphoreType.DMA,
        ],
    )
    def body(data_hbm, labels_hbm, out_hbm,
             dbuf, lbuf, sumtbl, cnttbl, stage, aggbuf,
             ctr_tbl, inv_tbl, cntv_tbl, vstage, varbuf, ostage,
             sh_stats, sh_var, sem0, sem1):
        cid = lax.axis_index("c")
        sid = lax.axis_index("s")
        iota = lax.iota(jnp.int32, L)
        zeros = jnp.zeros((L,), jnp.float32)
        ones = jnp.ones((L,), jnp.float32)
        sems = [sem0, sem1]

        def _start(i):
            slot = i % 2
            b = (i % (BPC * NCHUNK)) // NCHUNK
            ch = i % NCHUNK
            gb = cid * BPC + b
            row0 = pl.multiple_of((sid * PPT + ch * CH) // W, 8)
            h1 = pltpu.async_copy(data_hbm.at[gb, :, pl.ds(row0, RPC), :],
                                  dbuf.at[slot], sems[slot])
            h2 = pltpu.async_copy(labels_hbm.at[gb, pl.ds(row0, RPC), :],
                                  lbuf.at[slot], sems[slot])
            return (h1, h2)

        pending = [None] * NITEM
        pending[0] = _start(0)
        pending[1] = _start(1)

        def _finish(i):
            h1, h2 = pending[i]
            h1.wait()
            h2.wait()
            return i % 2

        for b in range(BPC):
            def _z_sum(r, _):
                sumtbl[pl.ds(r * L, L)] = zeros
                return 0
            lax.fori_loop(0, D * C, _z_sum, 0)

            def _z_cnt(r, _):
                cnttbl[pl.ds(r * L, L)] = zeros
                return 0
            lax.fori_loop(0, C, _z_cnt, 0)

            for ch in range(NCHUNK):
                i = b * NCHUNK + ch
                slot = _finish(i)

                @plsc.parallel_loop(0, NV, 1, unroll=4)
                def _vec1(v, slot=slot):
                    r = v // (W // L)
                    col = (v % (W // L)) * L
                    lab = lbuf[slot, r, pl.ds(col, L)]
                    idx = lab * L + iota
                    plsc.addupdate_scatter(cnttbl, [idx], ones)
                    for dim in range(D):
                        x = dbuf[slot, dim, r, pl.ds(col, L)]
                        plsc.addupdate_scatter(sumtbl,
                                               [idx + dim * (C * L)], x)
                if i + 2 < NITEM:
                    pending[i + 2] = _start(i + 2)

            scalars = [jnp.sum(sumtbl[pl.ds(r * L, L)]) for r in range(D * C)]
            scalars += [jnp.sum(cnttbl[pl.ds(c * L, L)]) for c in range(C)]
            for j in range(6):
                stage[pl.ds(j * L, L)] = _pack(scalars[j * L:(j + 1) * L], iota)
            pltpu.sync_copy(stage, sh_stats.at[pl.ds((b * NS + sid) * 96, 96)])

        plsc.subcore_barrier()

        k_list = []
        active_list = []
        pres_list = []
        for b in range(BPC):
            pltpu.sync_copy(sh_stats.at[pl.ds(b * NS * 96, NS * 96)], aggbuf)
            aggv = []
            for j in range(6):
                acc = zeros
                for t in range(NS):
                    acc = acc + aggbuf[pl.ds(t * 96 + j * L, L)]
                aggv.append(acc)

            csv = aggv[4]
            pres_vec = jnp.where(csv > 0.0, 1.0, 0.0)
            inv_vec = 1.0 / jnp.where(csv > 0.0, csv, 1.0)
            k = jnp.sum(pres_vec)
            active = jnp.where(k > 1.0, 1.0, 0.0)
            k_list.append(k)
            active_list.append(active)
            pres_list.append([pres_vec[c] for c in range(C)])

            inv_tbl[pl.ds(b * L, L)] = inv_vec
            cntv_tbl[pl.ds(b * L, L)] = csv
            inv = [inv_vec[c] for c in range(C)]
            for dim in range(D):
                ctr = [aggv[(dim * C + c) // L][(dim * C + c) % L] * inv[c]
                       for c in range(C)]
                ctr_tbl[pl.ds(b * (D * L) + dim * L, L)] = _pack(ctr, iota)

        for b in range(BPC):
            cbase = b * (D * L)

            def _z_var(r, _):
                cnttbl[pl.ds(r * L, L)] = zeros
                return 0
            lax.fori_loop(0, C, _z_var, 0)

            for ch in range(NCHUNK):
                i = BPC * NCHUNK + b * NCHUNK + ch
                slot = _finish(i)

                @plsc.parallel_loop(0, NV, 1, unroll=4)
                def _vec2(v, slot=slot, cbase=cbase):
                    r = v // (W // L)
                    col = (v % (W // L)) * L
                    lab = lbuf[slot, r, pl.ds(col, L)]
                    idx = lab * L + iota
                    dsq = jnp.zeros((L,), jnp.float32)
                    for dim in range(D):
                        x = dbuf[slot, dim, r, pl.ds(col, L)]
                        cv = plsc.load_gather(ctr_tbl,
                                              [lab + (cbase + dim * L)])
                        dd = x - cv
                        dsq = dsq + dd * dd
                    norm = _safe_norm(dsq)
                    t = jnp.maximum(norm - DELTA_VAR, 0.0)
                    plsc.addupdate_scatter(cnttbl, [idx], t * t)
                if i + 2 < NITEM:
                    pending[i + 2] = _start(i + 2)

            vstage[...] = _pack([jnp.sum(cnttbl[pl.ds(c * L, L)])
                                 for c in range(C)], iota)
            pltpu.sync_copy(vstage, sh_var.at[pl.ds((b * NS + sid) * L, L)])

        plsc.subcore_barrier()

        @pl.when(sid == 0)
        def _final():
            total = jnp.float32(0.0)
            for b in range(BPC):
                pltpu.sync_copy(sh_var.at[pl.ds(b * NS * L, NS * L)], varbuf)
                vsum = zeros
                for t in range(NS):
                    vsum = vsum + varbuf[pl.ds(t * L, L)]
                var_sum = jnp.sum(vsum * inv_tbl[pl.ds(b * L, L)])

                cbase = b * (D * L)
                cntv = cntv_tbl[pl.ds(b * L, L)]
                pres_vec = jnp.where(cntv > 0.0, 1.0, 0.0)
                k = k_list[b]
                pres = pres_list[b]

                dist_acc = jnp.float32(0.0)
                for ci in range(C):
                    dsq = jnp.zeros((L,), jnp.float32)
                    for dim in range(D):
                        cv = ctr_tbl[pl.ds(cbase + dim * L, L)]
                        dd = cv - jnp.broadcast_to(cv[ci], (L,))
                        dsq = dsq + dd * dd
                    dnorm = _safe_norm(dsq)
                    t = jnp.maximum(DELTA_DIST - dnorm, 0.0)
                    dist_acc = dist_acc + jnp.sum(t * t * pres_vec) * pres[ci]
                safe_k = jnp.where(k > 1.0, k, 2.0)
                dist_sum = dist_acc * _recip_s(2.0 * safe_k * (safe_k - 1.0))

                rsq = jnp.zeros((L,), jnp.float32)
                for dim in range(D):
                    cv = ctr_tbl[pl.ds(cbase + dim * L, L)]
                    rsq = rsq + cv * cv
                reg_sum = jnp.sum(_safe_norm(rsq) * pres_vec)
                reg_sum = reg_sum * _recip_s(jnp.where(k > 0.0, k, 1.0))

                total = total + active_list[b] * (var_sum + dist_sum + reg_sum)

            ostage[...] = jnp.broadcast_to(total, (L,))
            pltpu.sync_copy(ostage, out_hbm.at[pl.ds(cid * L, L)])

    return body


_sc_loss = jax.jit(_sc_loss_fn())


def kernel(data, labels):
    out = _sc_loss(data, labels.astype(jnp.int32))
    return (out[0] + out[L]) * (1.0 / B)
